# baseline (speedup 1.0000x reference)
import jax
import jax.numpy as jnp
from jax.experimental import pallas as pl
from jax.experimental.pallas import tpu as pltpu

_N = 16 * 3 * 512 * 512
_ROWS = 24576
_COLS = 512
_BLOCK_ROWS = 1024
_GRID = _ROWS // _BLOCK_ROWS


def _l1_mean_kernel(a_ref, b_ref, o_ref, acc_ref):
    i = pl.program_id(0)
    d = jnp.abs(a_ref[...] - b_ref[...])
    part = jnp.sum(d.reshape(_BLOCK_ROWS // 8, 8, _COLS), axis=0)

    @pl.when(i == 0)
    def _init():
        acc_ref[...] = part

    @pl.when(i > 0)
    def _acc():
        acc_ref[...] += part

    @pl.when(i == _GRID - 1)
    def _fin():
        o_ref[0, 0] = jnp.sum(acc_ref[...]) * (1.0 / _N)


def kernel(inp, tar, boxes, texts):
    a = inp.reshape(_ROWS, _COLS)
    b = tar.reshape(_ROWS, _COLS)
    out = pl.pallas_call(
        _l1_mean_kernel,
        grid=(_GRID,),
        in_specs=[
            pl.BlockSpec((_BLOCK_ROWS, _COLS), lambda i: (i, 0)),
            pl.BlockSpec((_BLOCK_ROWS, _COLS), lambda i: (i, 0)),
        ],
        out_specs=pl.BlockSpec(
            (1, 1), lambda i: (0, 0), memory_space=pltpu.SMEM
        ),
        out_shape=jax.ShapeDtypeStruct((1, 1), jnp.float32),
        scratch_shapes=[pltpu.VMEM((8, _COLS), jnp.float32)],
    )(a, b)
    return out[0, 0]


# speedup vs baseline: 1.0934x; 1.0934x over previous
import jax
import jax.numpy as jnp
from jax.experimental import pallas as pl
from jax.experimental.pallas import tpu as pltpu

_N = 16 * 3 * 512 * 512
_ROWS = 24576
_COLS = 512
_BLOCK_ROWS = 3072
_GRID = _ROWS // _BLOCK_ROWS


def _l1_mean_kernel(a_ref, b_ref, o_ref, acc_ref):
    i = pl.program_id(0)
    d = jnp.abs(a_ref[...] - b_ref[...])
    part = jnp.sum(d.reshape(_BLOCK_ROWS // 8, 8, _COLS), axis=0)

    @pl.when(i == 0)
    def _init():
        acc_ref[...] = part

    @pl.when(i > 0)
    def _acc():
        acc_ref[...] += part

    @pl.when(i == _GRID - 1)
    def _fin():
        o_ref[0, 0] = jnp.sum(acc_ref[...]) * (1.0 / _N)


def kernel(inp, tar, boxes, texts):
    a = inp.reshape(_ROWS, _COLS)
    b = tar.reshape(_ROWS, _COLS)
    out = pl.pallas_call(
        _l1_mean_kernel,
        grid=(_GRID,),
        in_specs=[
            pl.BlockSpec((_BLOCK_ROWS, _COLS), lambda i: (i, 0)),
            pl.BlockSpec((_BLOCK_ROWS, _COLS), lambda i: (i, 0)),
        ],
        out_specs=pl.BlockSpec(
            (1, 1), lambda i: (0, 0), memory_space=pltpu.SMEM
        ),
        out_shape=jax.ShapeDtypeStruct((1, 1), jnp.float32),
        scratch_shapes=[pltpu.VMEM((8, _COLS), jnp.float32)],
    )(a, b)
    return out[0, 0]


# speedup vs baseline: 1.1072x; 1.0126x over previous
import jax
import jax.numpy as jnp
from jax.experimental import pallas as pl
from jax.experimental.pallas import tpu as pltpu

_N = 16 * 3 * 512 * 512
_ROWS = 24576
_COLS = 512
_BLOCK_ROWS = 2048
_GRID = _ROWS // _BLOCK_ROWS


def _l1_mean_kernel(a_ref, b_ref, o_ref, acc_ref):
    i = pl.program_id(0)
    d = jnp.abs(a_ref[...] - b_ref[...])
    part = jnp.sum(d.reshape(_BLOCK_ROWS // 8, 8, _COLS), axis=0)

    @pl.when(i == 0)
    def _init():
        acc_ref[...] = part

    @pl.when(i > 0)
    def _acc():
        acc_ref[...] += part

    @pl.when(i == _GRID - 1)
    def _fin():
        o_ref[0, 0] = jnp.sum(acc_ref[...]) * (1.0 / _N)


def kernel(inp, tar, boxes, texts):
    a = inp.reshape(_ROWS, _COLS)
    b = tar.reshape(_ROWS, _COLS)
    out = pl.pallas_call(
        _l1_mean_kernel,
        grid=(_GRID,),
        in_specs=[
            pl.BlockSpec((_BLOCK_ROWS, _COLS), lambda i: (i, 0)),
            pl.BlockSpec((_BLOCK_ROWS, _COLS), lambda i: (i, 0)),
        ],
        out_specs=pl.BlockSpec(
            (1, 1), lambda i: (0, 0), memory_space=pltpu.SMEM
        ),
        out_shape=jax.ShapeDtypeStruct((1, 1), jnp.float32),
        scratch_shapes=[pltpu.VMEM((8, _COLS), jnp.float32)],
    )(a, b)
    return out[0, 0]


# speedup vs baseline: 1.1246x; 1.0158x over previous
import jax
import jax.numpy as jnp
from jax.experimental import pallas as pl
from jax.experimental.pallas import tpu as pltpu

_N = 16 * 3 * 512 * 512
_ROWS = 24576
_COLS = 512
_K = 4
_STEPS = 12
_BR = _ROWS // (_K * _STEPS)


def _l1_mean_kernel(*refs):
    a_refs = refs[:_K]
    b_refs = refs[_K:2 * _K]
    o_ref = refs[2 * _K]
    acc_ref = refs[2 * _K + 1]
    i = pl.program_id(0)

    total = jnp.zeros((8, _COLS), jnp.float32)
    for k in range(_K):
        d = jnp.abs(a_refs[k][...] - b_refs[k][...])
        total = total + jnp.sum(d.reshape(_BR // 8, 8, _COLS), axis=0)

    @pl.when(i == 0)
    def _init():
        acc_ref[...] = total

    @pl.when(i > 0)
    def _acc():
        acc_ref[...] += total

    @pl.when(i == _STEPS - 1)
    def _fin():
        o_ref[0, 0] = jnp.sum(acc_ref[...]) * (1.0 / _N)


def _make_spec(k):
    return pl.BlockSpec((_BR, _COLS), lambda i, k=k: (k * _STEPS + i, 0))


def kernel(inp, tar, boxes, texts):
    a = inp.reshape(_ROWS, _COLS)
    b = tar.reshape(_ROWS, _COLS)
    in_specs = [_make_spec(k) for k in range(_K)] * 2
    out = pl.pallas_call(
        _l1_mean_kernel,
        grid=(_STEPS,),
        in_specs=in_specs,
        out_specs=pl.BlockSpec(
            (1, 1), lambda i: (0, 0), memory_space=pltpu.SMEM
        ),
        out_shape=jax.ShapeDtypeStruct((1, 1), jnp.float32),
        scratch_shapes=[pltpu.VMEM((8, _COLS), jnp.float32)],
    )(*([a] * _K + [b] * _K))
    return out[0, 0]
